# baseline (speedup 1.0000x reference)
import functools

import numpy as np
import jax
import jax.numpy as jnp
from jax import lax
from jax.experimental import pallas as pl
from jax.experimental.pallas import tpu as pltpu
from jax.experimental.pallas import tpu_sc as plsc

_NC = 2
_NS = 16
_NW = _NC * _NS
_LANES = 16


def _pe_table(seq_len, d_model):
    pos = np.arange(seq_len, dtype=np.float32)[:, None]
    i = np.arange(0, d_model, 2, dtype=np.float32)
    div = np.power(10000.0, i / d_model)
    pe = np.zeros((seq_len, d_model), dtype=np.float32)
    pe[:, 0::2] = np.sin(pos / div)
    pe[:, 1::2] = np.cos(pos / div)
    return jnp.asarray(pe)


def kernel(x, tok_table):
    B, S = x.shape
    V, D = tok_table.shape
    N = B * S
    PW = S // _NW
    C = 16
    Q = PW // C
    NCH = Q * B
    NR = 4
    assert S % _NW == 0 and PW % C == 0 and NCH >= 4
    assert B == NR and Q % 2 == 0

    pe = _pe_table(S, D)
    idx = x.reshape(N)
    mesh = plsc.VectorSubcoreMesh(core_axis_name="c", subcore_axis_name="s")

    @functools.partial(
        pl.kernel,
        mesh=mesh,
        out_type=jax.ShapeDtypeStruct((N, D), jnp.float32),
        scratch_types=(
            [pltpu.VMEM((B * PW,), jnp.int32)]
            + [pltpu.VMEM((C, D), jnp.float32) for _ in range(NR)]
            + [pltpu.VMEM((C, D), jnp.float32) for _ in range(2)]
            + [pltpu.SemaphoreType.DMA for _ in range(NR * 2 + 2)]
        ),
    )
    def emb(tab_hbm, idx_hbm, pe_hbm, out_hbm, idx_v,
            g0, g1, g2, g3, p0, p1,
            sg0, sg1, sg2, sg3, ss0, ss1, ss2, ss3, sp0, sp1):
        g = (g0, g1, g2, g3)
        sg = (sg0, sg1, sg2, sg3)
        ss = (ss0, ss1, ss2, ss3)
        p = (p0, p1)
        sp = (sp0, sp1)

        wid = lax.axis_index("s") * _NC + lax.axis_index("c")
        base_pos = wid * PW
        for b in range(B):
            pltpu.sync_copy(idx_hbm.at[pl.ds(b * S + base_pos, PW)],
                            idx_v.at[pl.ds(b * PW, PW)])

        def gather_desc(c, r):
            return pltpu.make_async_copy(
                tab_hbm.at[idx_v.at[pl.ds((c % B) * PW + (c // B) * C, C)]],
                g[r], sg[r])

        def pe_desc(q, qb):
            return pltpu.make_async_copy(
                pe_hbm.at[pl.ds(base_pos + q * C, C)], p[qb], sp[qb])

        def store_desc(c, r):
            return pltpu.make_async_copy(
                g[r],
                out_hbm.at[pl.ds((c % B) * S + base_pos + (c // B) * C, C)],
                ss[r])

        gather_desc(0, 0).start()
        gather_desc(1, 1).start()
        pe_desc(0, 0).start()
        pe_desc(1, 1).start()

        @pl.loop(0, Q, step=2)
        def _(qj):
            for qq in range(2):
                qb = qq
                q = qj + qq
                pe_desc(q, qb).wait()
                for b in range(B):
                    c = q * B + b
                    gather_desc(c, b).wait()

                    @pl.loop(0, C)
                    def _(r_):
                        for u in range(D // _LANES):
                            sl = pl.ds(u * _LANES, _LANES)
                            plsc.addupdate(g[b].at[r_, sl],
                                           p[qb].at[r_, sl][...])

                    store_desc(c, b).start()

                    @pl.when(c >= 2)
                    def _():
                        store_desc(c - 2, (b + 2) % NR).wait()

                    @pl.when(c + 2 < NCH)
                    def _():
                        gather_desc(c + 2, (b + 2) % NR).start()

                @pl.when(q + 2 < Q)
                def _():
                    pe_desc(q + 2, qb).start()

        store_desc(NCH - 2, (NCH - 2) % NR).wait()
        store_desc(NCH - 1, (NCH - 1) % NR).wait()

    out = emb(tok_table, idx, pe)
    return out.reshape(B, S, D)


# speedup vs baseline: 1.0425x; 1.0425x over previous
import functools

import numpy as np
import jax
import jax.numpy as jnp
from jax import lax
from jax.experimental import pallas as pl
from jax.experimental.pallas import tpu as pltpu
from jax.experimental.pallas import tpu_sc as plsc

_NC = 2
_NS = 16
_NW = _NC * _NS
_LANES = 16


def _pe_table(seq_len, d_model):
    pos = np.arange(seq_len, dtype=np.float32)[:, None]
    i = np.arange(0, d_model, 2, dtype=np.float32)
    div = np.power(10000.0, i / d_model)
    pe = np.zeros((seq_len, d_model), dtype=np.float32)
    pe[:, 0::2] = np.sin(pos / div)
    pe[:, 1::2] = np.cos(pos / div)
    return jnp.asarray(pe)


def kernel(x, tok_table):
    B, S = x.shape
    V, D = tok_table.shape
    N = B * S
    PW = S // _NW
    C = 16
    Q = PW // C
    NCH = Q * B
    NR = 4
    assert S % _NW == 0 and PW % C == 0 and NCH >= 4
    assert B == NR and Q % 2 == 0

    pe = _pe_table(S, D)
    idx = x.reshape(N)
    mesh = plsc.VectorSubcoreMesh(core_axis_name="c", subcore_axis_name="s")

    @functools.partial(
        pl.kernel,
        mesh=mesh,
        out_type=jax.ShapeDtypeStruct((N, D), jnp.float32),
        scratch_types=(
            [pltpu.VMEM((B * PW,), jnp.int32)]
            + [pltpu.VMEM((C, D), jnp.float32) for _ in range(NR)]
            + [pltpu.VMEM((C, D), jnp.float32) for _ in range(2)]
            + [pltpu.SemaphoreType.DMA for _ in range(NR * 2 + 2)]
        ),
    )
    def emb(tab_hbm, idx_hbm, pe_hbm, out_hbm, idx_v,
            g0, g1, g2, g3, p0, p1,
            sg0, sg1, sg2, sg3, ss0, ss1, ss2, ss3, sp0, sp1):
        g = (g0, g1, g2, g3)
        sg = (sg0, sg1, sg2, sg3)
        ss = (ss0, ss1, ss2, ss3)
        p = (p0, p1)
        sp = (sp0, sp1)

        wid = lax.axis_index("s") * _NC + lax.axis_index("c")
        base_pos = wid * PW
        for b in range(B):
            pltpu.sync_copy(idx_hbm.at[pl.ds(b * S + base_pos, PW)],
                            idx_v.at[pl.ds(b * PW, PW)])

        def gather_desc(c, r):
            return pltpu.make_async_copy(
                tab_hbm.at[idx_v.at[pl.ds((c % B) * PW + (c // B) * C, C)]],
                g[r], sg[r])

        def pe_desc(q, qb):
            return pltpu.make_async_copy(
                pe_hbm.at[pl.ds(base_pos + q * C, C)], p[qb], sp[qb])

        def store_desc(c, r):
            return pltpu.make_async_copy(
                g[r],
                out_hbm.at[pl.ds((c % B) * S + base_pos + (c // B) * C, C)],
                ss[r])

        gather_desc(0, 0).start()
        gather_desc(1, 1).start()
        pe_desc(0, 0).start()
        pe_desc(1, 1).start()

        @pl.loop(0, Q, step=2)
        def _(qj):
            for qq in range(2):
                qb = qq
                q = qj + qq
                pe_desc(q, qb).wait()
                for b in range(B):
                    c = q * B + b
                    gather_desc(c, b).wait()

                    @plsc.parallel_loop(0, C * D, step=4 * _LANES, unroll=4)
                    def _(e):
                        r_ = e // D
                        for u in range(4):
                            sl = pl.ds(e % D + u * _LANES, _LANES)
                            plsc.addupdate(g[b].at[r_, sl],
                                           p[qb].at[r_, sl][...])

                    store_desc(c, b).start()

                    @pl.when(c >= 2)
                    def _():
                        store_desc(c - 2, (b + 2) % NR).wait()

                    @pl.when(c + 2 < NCH)
                    def _():
                        gather_desc(c + 2, (b + 2) % NR).start()

                @pl.when(q + 2 < Q)
                def _():
                    pe_desc(q + 2, qb).start()

        store_desc(NCH - 2, (NCH - 2) % NR).wait()
        store_desc(NCH - 1, (NCH - 1) % NR).wait()

    out = emb(tok_table, idx, pe)
    return out.reshape(B, S, D)


# speedup vs baseline: 1.1163x; 1.0708x over previous
import functools

import numpy as np
import jax
import jax.numpy as jnp
from jax import lax
from jax.experimental import pallas as pl
from jax.experimental.pallas import tpu as pltpu
from jax.experimental.pallas import tpu_sc as plsc

_NC = 2
_NS = 16
_NW = _NC * _NS
_LANES = 16


def _pe_table(seq_len, d_model):
    pos = np.arange(seq_len, dtype=np.float32)[:, None]
    i = np.arange(0, d_model, 2, dtype=np.float32)
    div = np.power(10000.0, i / d_model)
    pe = np.zeros((seq_len, d_model), dtype=np.float32)
    pe[:, 0::2] = np.sin(pos / div)
    pe[:, 1::2] = np.cos(pos / div)
    return jnp.asarray(pe)


def kernel(x, tok_table):
    B, S = x.shape
    V, D = tok_table.shape
    N = B * S
    PW = S // _NW
    C = 16
    Q = PW // C
    NCH = Q * B
    NR = 4
    assert S % _NW == 0 and PW % C == 0 and NCH >= 4
    assert B == NR and Q % 2 == 0

    pe = _pe_table(S, D)
    idx = x.reshape(N)
    mesh = plsc.VectorSubcoreMesh(core_axis_name="c", subcore_axis_name="s")

    @functools.partial(
        pl.kernel,
        mesh=mesh,
        out_type=jax.ShapeDtypeStruct((N, D), jnp.float32),
        scratch_types=(
            [pltpu.VMEM((B * PW,), jnp.int32)]
            + [pltpu.VMEM((C, D), jnp.float32) for _ in range(NR)]
            + [pltpu.VMEM((C, D), jnp.float32) for _ in range(2)]
            + [pltpu.SemaphoreType.DMA for _ in range(NR * 2 + 2)]
        ),
    )
    def emb(tab_hbm, idx_hbm, pe_hbm, out_hbm, idx_v,
            g0, g1, g2, g3, p0, p1,
            sg0, sg1, sg2, sg3, ss0, ss1, ss2, ss3, sp0, sp1):
        g = (g0, g1, g2, g3)
        sg = (sg0, sg1, sg2, sg3)
        ss = (ss0, ss1, ss2, ss3)
        p = (p0, p1)
        sp = (sp0, sp1)

        wid = lax.axis_index("s") * _NC + lax.axis_index("c")
        base_pos = wid * PW

        def idx_desc(b):
            return pltpu.make_async_copy(
                idx_hbm.at[pl.ds(b * S + base_pos, PW)],
                idx_v.at[pl.ds(b * PW, PW)], ss[b])

        def gather_desc(c, r):
            return pltpu.make_async_copy(
                tab_hbm.at[idx_v.at[pl.ds((c % B) * PW + (c // B) * C, C)]],
                g[r], sg[r])

        def pe_desc(q, qb):
            return pltpu.make_async_copy(
                pe_hbm.at[pl.ds(base_pos + q * C, C)], p[qb], sp[qb])

        def store_desc(c, r):
            return pltpu.make_async_copy(
                g[r],
                out_hbm.at[pl.ds((c % B) * S + base_pos + (c // B) * C, C)],
                ss[r])

        pe_desc(0, 0).start()
        pe_desc(1, 1).start()
        for b in range(B):
            idx_desc(b).start()
        idx_desc(0).wait()
        gather_desc(0, 0).start()
        idx_desc(1).wait()
        gather_desc(1, 1).start()
        idx_desc(2).wait()
        idx_desc(3).wait()

        @pl.loop(0, Q, step=2)
        def _(qj):
            for qq in range(2):
                qb = qq
                q = qj + qq
                pe_desc(q, qb).wait()
                for b in range(B):
                    c = q * B + b
                    gather_desc(c, b).wait()

                    @pl.when(c >= 2)
                    def _():
                        store_desc(c - 2, (b + 2) % NR).wait()

                    @pl.when(c + 2 < NCH)
                    def _():
                        gather_desc(c + 2, (b + 2) % NR).start()

                    @plsc.parallel_loop(0, C * D, step=4 * _LANES, unroll=4)
                    def _(e):
                        r_ = e // D
                        for u in range(4):
                            sl = pl.ds(e % D + u * _LANES, _LANES)
                            plsc.addupdate(g[b].at[r_, sl],
                                           p[qb].at[r_, sl][...])

                    store_desc(c, b).start()

                @pl.when(q + 2 < Q)
                def _():
                    pe_desc(q + 2, qb).start()

        store_desc(NCH - 2, (NCH - 2) % NR).wait()
        store_desc(NCH - 1, (NCH - 1) % NR).wait()

    out = emb(tok_table, idx, pe)
    return out.reshape(B, S, D)
